# Initial kernel scaffold; baseline (speedup 1.0000x reference)
import jax
import jax.numpy as jnp
from jax.experimental import pallas as pl


def kernel(x, emb_table, fc_w, fc_b):
    raise NotImplementedError("write your pallas kernel here")


# speedup vs baseline: 6.6240x; 6.6240x over previous
import functools

import jax
import jax.numpy as jnp
from jax import lax
from jax.experimental import pallas as pl
from jax.experimental.pallas import tpu as pltpu
from jax.experimental.pallas import tpu_sc as plsc

NUM_EMBED = 100000
EMBED_DIM = 768
OUT_DIM = 256
HALF = OUT_DIM // 2
BATCH = 4096
SEQ = 200


_MM_BLOCK = 1000


def _mm_body(e_ref, wlo_ref, whi_ref, b_ref, olo_ref, ohi_ref):
    e = e_ref[...]
    olo_ref[...] = (
        jnp.dot(e, wlo_ref[...], preferred_element_type=jnp.float32)
        + b_ref[0:1, :HALF]
    )
    ohi_ref[...] = (
        jnp.dot(e, whi_ref[...], preferred_element_type=jnp.float32)
        + b_ref[0:1, HALF:]
    )


def _project_table(emb_table, fc_wt_lo, fc_wt_hi, fc_b2d):
    return pl.pallas_call(
        _mm_body,
        grid=(NUM_EMBED // _MM_BLOCK,),
        in_specs=[
            pl.BlockSpec((_MM_BLOCK, EMBED_DIM), lambda i: (i, 0)),
            pl.BlockSpec((EMBED_DIM, HALF), lambda i: (0, 0)),
            pl.BlockSpec((EMBED_DIM, HALF), lambda i: (0, 0)),
            pl.BlockSpec((1, OUT_DIM), lambda i: (0, 0)),
        ],
        out_specs=[
            pl.BlockSpec((_MM_BLOCK, HALF), lambda i: (i, 0)),
            pl.BlockSpec((_MM_BLOCK, HALF), lambda i: (i, 0)),
        ],
        out_shape=[
            jax.ShapeDtypeStruct((NUM_EMBED, HALF), jnp.float32),
            jax.ShapeDtypeStruct((NUM_EMBED, HALF), jnp.float32),
        ],
    )(emb_table, fc_wt_lo, fc_wt_hi, fc_b2d)


_NW = 32
_ROWS_PER_W = BATCH // _NW
_CHUNK = 100
_CHUNKS_PER_ROW = SEQ // _CHUNK
_CHUNKS_PER_W = _ROWS_PER_W * _CHUNKS_PER_ROW
_IDX_STAGE = 128
_NG = HALF // 16
_INV_SEQ = 1.0 / SEQ


def _gather_mean_body(plo_hbm, phi_hbm, xr_hbm, out_hbm, idx_v, blo_a, bhi_a,
                      blo_b, bhi_b, out_v, sem_a, sem_b):
    wid = lax.axis_index("s") * 2 + lax.axis_index("c")

    def accumulate(blo, bhi, acc):
        def body(i, acc):
            lo = tuple(
                acc[g] + blo[i, pl.ds(16 * g, 16)] for g in range(_NG)
            )
            hi = tuple(
                acc[_NG + g] + bhi[i, pl.ds(16 * g, 16)] for g in range(_NG)
            )
            return lo + hi
        return lax.fori_loop(0, _CHUNK, body, acc)

    zeros = tuple(jnp.zeros((16,), jnp.float32) for _ in range(2 * _NG))

    def stage_body(s, _):
        pltpu.sync_copy(
            xr_hbm.at[pl.ds(wid * _CHUNKS_PER_W + s * _IDX_STAGE, _IDX_STAGE)],
            idx_v)

        def row_body(r, _):
            pltpu.async_copy(plo_hbm.at[idx_v.at[2 * r]], blo_a, sem_a)
            pltpu.async_copy(phi_hbm.at[idx_v.at[2 * r]], bhi_a, sem_a)
            pltpu.async_copy(plo_hbm.at[idx_v.at[2 * r + 1]], blo_b, sem_b)
            pltpu.async_copy(phi_hbm.at[idx_v.at[2 * r + 1]], bhi_b, sem_b)

            pltpu.make_async_copy(plo_hbm.at[idx_v.at[2 * r]], blo_a,
                                  sem_a).wait()
            pltpu.make_async_copy(phi_hbm.at[idx_v.at[2 * r]], bhi_a,
                                  sem_a).wait()
            acc = accumulate(blo_a, bhi_a, zeros)

            pltpu.make_async_copy(plo_hbm.at[idx_v.at[2 * r + 1]], blo_b,
                                  sem_b).wait()
            pltpu.make_async_copy(phi_hbm.at[idx_v.at[2 * r + 1]], bhi_b,
                                  sem_b).wait()
            acc = accumulate(blo_b, bhi_b, acc)

            out_r = s * (_IDX_STAGE // 2) + r
            for g in range(_NG):
                out_v[out_r, pl.ds(16 * g, 16)] = acc[g] * _INV_SEQ
            for g in range(_NG):
                out_v[out_r, pl.ds(HALF + 16 * g, 16)] = (
                    acc[_NG + g] * _INV_SEQ)
            return 0

        lax.fori_loop(0, _IDX_STAGE // 2, row_body, 0)
        return 0

    lax.fori_loop(0, _CHUNKS_PER_W // _IDX_STAGE, stage_body, 0)

    pltpu.sync_copy(out_v, out_hbm.at[pl.ds(wid * _ROWS_PER_W, _ROWS_PER_W)])


def _gather_mean(p_lo, p_hi, x_chunks):
    mesh = plsc.VectorSubcoreMesh(core_axis_name="c", subcore_axis_name="s")
    run = functools.partial(
        pl.kernel,
        mesh=mesh,
        out_type=jax.ShapeDtypeStruct((BATCH, OUT_DIM), jnp.float32),
        scratch_types=[
            pltpu.VMEM((_IDX_STAGE, _CHUNK), jnp.int32),
            pltpu.VMEM((_CHUNK, HALF), jnp.float32),
            pltpu.VMEM((_CHUNK, HALF), jnp.float32),
            pltpu.VMEM((_CHUNK, HALF), jnp.float32),
            pltpu.VMEM((_CHUNK, HALF), jnp.float32),
            pltpu.VMEM((_ROWS_PER_W, OUT_DIM), jnp.float32),
            pltpu.SemaphoreType.DMA,
            pltpu.SemaphoreType.DMA,
        ],
    )(_gather_mean_body)
    return run(p_lo, p_hi, x_chunks)


def kernel(x, emb_table, fc_w, fc_b):
    fc_wt = fc_w.T
    p_lo, p_hi = _project_table(emb_table, fc_wt[:, :HALF], fc_wt[:, HALF:],
                                fc_b.reshape(1, OUT_DIM))
    x_chunks = x.astype(jnp.int32).reshape(BATCH * SEQ // _CHUNK, _CHUNK)
    return _gather_mean(p_lo, p_hi, x_chunks)


# speedup vs baseline: 9.1860x; 1.3868x over previous
import functools

import jax
import jax.numpy as jnp
from jax import lax
from jax.experimental import pallas as pl
from jax.experimental.pallas import tpu as pltpu
from jax.experimental.pallas import tpu_sc as plsc

NUM_EMBED = 100000
EMBED_DIM = 768
OUT_DIM = 256
HALF = OUT_DIM // 2
BATCH = 4096
SEQ = 200


_MM_BLOCK = 1000


def _mm_body(e_ref, wlo_ref, whi_ref, b_ref, olo_ref, ohi_ref):
    e = e_ref[...]
    olo_ref[...] = (
        jnp.dot(e, wlo_ref[...], preferred_element_type=jnp.float32)
        + b_ref[0:1, :HALF]
    )
    ohi_ref[...] = (
        jnp.dot(e, whi_ref[...], preferred_element_type=jnp.float32)
        + b_ref[0:1, HALF:]
    )


def _project_table(emb_table, fc_wt_lo, fc_wt_hi, fc_b2d):
    return pl.pallas_call(
        _mm_body,
        grid=(NUM_EMBED // _MM_BLOCK,),
        in_specs=[
            pl.BlockSpec((_MM_BLOCK, EMBED_DIM), lambda i: (i, 0)),
            pl.BlockSpec((EMBED_DIM, HALF), lambda i: (0, 0)),
            pl.BlockSpec((EMBED_DIM, HALF), lambda i: (0, 0)),
            pl.BlockSpec((1, OUT_DIM), lambda i: (0, 0)),
        ],
        out_specs=[
            pl.BlockSpec((_MM_BLOCK, HALF), lambda i: (i, 0)),
            pl.BlockSpec((_MM_BLOCK, HALF), lambda i: (i, 0)),
        ],
        out_shape=[
            jax.ShapeDtypeStruct((NUM_EMBED, HALF), jnp.float32),
            jax.ShapeDtypeStruct((NUM_EMBED, HALF), jnp.float32),
        ],
    )(emb_table, fc_wt_lo, fc_wt_hi, fc_b2d)


_NW = 32
_ROWS_PER_W = BATCH // _NW
_CHUNK = 100
_CHUNKS_PER_ROW = SEQ // _CHUNK
_CHUNKS_PER_W = _ROWS_PER_W * _CHUNKS_PER_ROW
_NG = HALF // 16
_INV_SEQ = 1.0 / SEQ


def _gather_mean_body(plo_hbm, phi_hbm, xr_hbm, out_hbm, idx_v, blo_a, bhi_a,
                      blo_b, bhi_b, out_v, sem_a, sem_b):
    wid = lax.axis_index("s") * 2 + lax.axis_index("c")

    pltpu.sync_copy(xr_hbm.at[pl.ds(wid * _CHUNKS_PER_W, _CHUNKS_PER_W)],
                    idx_v)

    def start_chunk(c, blo, bhi, sem):
        pltpu.async_copy(plo_hbm.at[idx_v.at[c]], blo, sem)
        pltpu.async_copy(phi_hbm.at[idx_v.at[c]], bhi, sem)

    def wait_chunk(c, blo, bhi, sem):
        pltpu.make_async_copy(plo_hbm.at[idx_v.at[c]], blo, sem).wait()
        pltpu.make_async_copy(phi_hbm.at[idx_v.at[c]], bhi, sem).wait()

    def accumulate(blo, bhi, acc):
        def body(i, acc):
            lo = tuple(
                acc[g] + blo[i, pl.ds(16 * g, 16)] for g in range(_NG)
            )
            hi = tuple(
                acc[_NG + g] + bhi[i, pl.ds(16 * g, 16)] for g in range(_NG)
            )
            return lo + hi
        return lax.fori_loop(0, _CHUNK, body, acc)

    zeros = tuple(jnp.zeros((16,), jnp.float32) for _ in range(2 * _NG))

    def store_row(row, acc):
        for g in range(_NG):
            out_v[row, pl.ds(16 * g, 16)] = acc[g] * _INV_SEQ
        for g in range(_NG):
            out_v[row, pl.ds(HALF + 16 * g, 16)] = acc[_NG + g] * _INV_SEQ

    start_chunk(0, blo_a, bhi_a, sem_a)

    def pair_body(r, _):
        c_a = 2 * r
        c_b = c_a + 1

        start_chunk(c_b, blo_b, bhi_b, sem_b)
        wait_chunk(c_a, blo_a, bhi_a, sem_a)
        acc = accumulate(blo_a, bhi_a, zeros)

        @pl.when(r < _ROWS_PER_W - 1)
        def _():
            start_chunk(c_a + 2, blo_a, bhi_a, sem_a)

        wait_chunk(c_b, blo_b, bhi_b, sem_b)
        acc = accumulate(blo_b, bhi_b, acc)
        store_row(r, acc)
        return 0

    lax.fori_loop(0, _ROWS_PER_W, pair_body, 0)

    pltpu.sync_copy(out_v, out_hbm.at[pl.ds(wid * _ROWS_PER_W, _ROWS_PER_W)])


def _gather_mean(p_lo, p_hi, x_chunks):
    mesh = plsc.VectorSubcoreMesh(core_axis_name="c", subcore_axis_name="s")
    run = functools.partial(
        pl.kernel,
        mesh=mesh,
        out_type=jax.ShapeDtypeStruct((BATCH, OUT_DIM), jnp.float32),
        scratch_types=[
            pltpu.VMEM((_CHUNKS_PER_W, _CHUNK), jnp.int32),
            pltpu.VMEM((_CHUNK, HALF), jnp.float32),
            pltpu.VMEM((_CHUNK, HALF), jnp.float32),
            pltpu.VMEM((_CHUNK, HALF), jnp.float32),
            pltpu.VMEM((_CHUNK, HALF), jnp.float32),
            pltpu.VMEM((_ROWS_PER_W, OUT_DIM), jnp.float32),
            pltpu.SemaphoreType.DMA,
            pltpu.SemaphoreType.DMA,
        ],
    )(_gather_mean_body)
    return run(p_lo, p_hi, x_chunks)


def kernel(x, emb_table, fc_w, fc_b):
    fc_wt = fc_w.T
    p_lo, p_hi = _project_table(emb_table, fc_wt[:, :HALF], fc_wt[:, HALF:],
                                fc_b.reshape(1, OUT_DIM))
    x_chunks = x.astype(jnp.int32).reshape(BATCH * SEQ // _CHUNK, _CHUNK)
    return _gather_mean(p_lo, p_hi, x_chunks)


# speedup vs baseline: 11.8201x; 1.2868x over previous
import functools

import jax
import jax.numpy as jnp
from jax import lax
from jax.experimental import pallas as pl
from jax.experimental.pallas import tpu as pltpu
from jax.experimental.pallas import tpu_sc as plsc

NUM_EMBED = 100000
EMBED_DIM = 768
OUT_DIM = 256
HALF = OUT_DIM // 2
BATCH = 4096
SEQ = 200


_MM_BLOCK = 1000


def _pack_bf16_pair(lo, hi):
    lo_bits = lax.bitcast_convert_type(
        lo.astype(jnp.bfloat16).astype(jnp.float32), jnp.int32)
    hi_bits = lax.bitcast_convert_type(
        hi.astype(jnp.bfloat16).astype(jnp.float32), jnp.int32)
    return lax.shift_right_logical(lo_bits, 16) | (
        hi_bits & jnp.int32(-65536))


def _mm_body(e_ref, wlo_ref, whi_ref, b_ref, o_ref):
    e = e_ref[...]
    lo = (jnp.dot(e, wlo_ref[...], preferred_element_type=jnp.float32)
          + b_ref[0:1, :HALF])
    hi = (jnp.dot(e, whi_ref[...], preferred_element_type=jnp.float32)
          + b_ref[0:1, HALF:])
    o_ref[...] = _pack_bf16_pair(lo, hi)


def _project_table(emb_table, fc_wt_lo, fc_wt_hi, fc_b2d):
    return pl.pallas_call(
        _mm_body,
        grid=(NUM_EMBED // _MM_BLOCK,),
        in_specs=[
            pl.BlockSpec((_MM_BLOCK, EMBED_DIM), lambda i: (i, 0)),
            pl.BlockSpec((EMBED_DIM, HALF), lambda i: (0, 0)),
            pl.BlockSpec((EMBED_DIM, HALF), lambda i: (0, 0)),
            pl.BlockSpec((1, OUT_DIM), lambda i: (0, 0)),
        ],
        out_specs=pl.BlockSpec((_MM_BLOCK, HALF), lambda i: (i, 0)),
        out_shape=jax.ShapeDtypeStruct((NUM_EMBED, HALF), jnp.int32),
    )(emb_table, fc_wt_lo, fc_wt_hi, fc_b2d)


_NW = 32
_ROWS_PER_W = BATCH // _NW
_CHUNK = 100
_CHUNKS_PER_ROW = SEQ // _CHUNK
_CHUNKS_PER_W = _ROWS_PER_W * _CHUNKS_PER_ROW
_NG = HALF // 16
_INV_SEQ = 1.0 / SEQ


def _gather_mean_body(p_hbm, xr_hbm, out_hbm, idx_v, buf_a, buf_b, out_v,
                      sem_a, sem_b):
    wid = lax.axis_index("s") * 2 + lax.axis_index("c")

    pltpu.sync_copy(xr_hbm.at[pl.ds(wid * _CHUNKS_PER_W, _CHUNKS_PER_W)],
                    idx_v)

    def accumulate(buf, acc):
        def body(i, acc):
            acc = list(acc)
            for g in range(_NG):
                packed = buf[i, pl.ds(16 * g, 16)]
                acc[g] = acc[g] + lax.bitcast_convert_type(
                    packed << 16, jnp.float32)
                acc[_NG + g] = acc[_NG + g] + lax.bitcast_convert_type(
                    packed & jnp.int32(-65536), jnp.float32)
            return tuple(acc)
        return lax.fori_loop(0, _CHUNK, body, acc)

    zeros = tuple(jnp.zeros((16,), jnp.float32) for _ in range(2 * _NG))

    def store_row(row, acc):
        for g in range(_NG):
            out_v[row, pl.ds(16 * g, 16)] = acc[g] * _INV_SEQ
        for g in range(_NG):
            out_v[row, pl.ds(HALF + 16 * g, 16)] = acc[_NG + g] * _INV_SEQ

    pltpu.async_copy(p_hbm.at[idx_v.at[0]], buf_a, sem_a)

    def pair_body(r, _):
        c_a = 2 * r
        c_b = c_a + 1

        pltpu.async_copy(p_hbm.at[idx_v.at[c_b]], buf_b, sem_b)
        pltpu.make_async_copy(p_hbm.at[idx_v.at[c_a]], buf_a, sem_a).wait()
        acc = accumulate(buf_a, zeros)

        @pl.when(r < _ROWS_PER_W - 1)
        def _():
            pltpu.async_copy(p_hbm.at[idx_v.at[c_a + 2]], buf_a, sem_a)

        pltpu.make_async_copy(p_hbm.at[idx_v.at[c_b]], buf_b, sem_b).wait()
        acc = accumulate(buf_b, acc)
        store_row(r, acc)
        return 0

    lax.fori_loop(0, _ROWS_PER_W, pair_body, 0)

    pltpu.sync_copy(out_v, out_hbm.at[pl.ds(wid * _ROWS_PER_W, _ROWS_PER_W)])


def _gather_mean(p, x_chunks):
    mesh = plsc.VectorSubcoreMesh(core_axis_name="c", subcore_axis_name="s")
    run = functools.partial(
        pl.kernel,
        mesh=mesh,
        out_type=jax.ShapeDtypeStruct((BATCH, OUT_DIM), jnp.float32),
        scratch_types=[
            pltpu.VMEM((_CHUNKS_PER_W, _CHUNK), jnp.int32),
            pltpu.VMEM((_CHUNK, HALF), jnp.int32),
            pltpu.VMEM((_CHUNK, HALF), jnp.int32),
            pltpu.VMEM((_ROWS_PER_W, OUT_DIM), jnp.float32),
            pltpu.SemaphoreType.DMA,
            pltpu.SemaphoreType.DMA,
        ],
    )(_gather_mean_body)
    return run(p, x_chunks)


def kernel(x, emb_table, fc_w, fc_b):
    fc_wt = fc_w.T
    p = _project_table(emb_table, fc_wt[:, :HALF], fc_wt[:, HALF:],
                       fc_b.reshape(1, OUT_DIM))
    x_chunks = x.astype(jnp.int32).reshape(BATCH * SEQ // _CHUNK, _CHUNK)
    return _gather_mean(p, x_chunks)


# speedup vs baseline: 12.3744x; 1.0469x over previous
import functools

import jax
import jax.numpy as jnp
from jax import lax
from jax.experimental import pallas as pl
from jax.experimental.pallas import tpu as pltpu
from jax.experimental.pallas import tpu_sc as plsc

NUM_EMBED = 100000
EMBED_DIM = 768
OUT_DIM = 256
HALF = OUT_DIM // 2
BATCH = 4096
SEQ = 200


_MM_BLOCK = 1000


def _pack_bf16_pair(lo, hi):
    lo_bits = lax.bitcast_convert_type(
        lo.astype(jnp.bfloat16).astype(jnp.float32), jnp.int32)
    hi_bits = lax.bitcast_convert_type(
        hi.astype(jnp.bfloat16).astype(jnp.float32), jnp.int32)
    return lax.shift_right_logical(lo_bits, 16) | (
        hi_bits & jnp.int32(-65536))


def _mm_body(e_ref, wlo_ref, whi_ref, b_ref, o_ref):
    e = e_ref[...]
    lo = (jnp.dot(e, wlo_ref[...], preferred_element_type=jnp.float32)
          + b_ref[0:1, :HALF])
    hi = (jnp.dot(e, whi_ref[...], preferred_element_type=jnp.float32)
          + b_ref[0:1, HALF:])
    o_ref[...] = _pack_bf16_pair(lo, hi)


def _project_table(emb_table, fc_wt_lo, fc_wt_hi, fc_b2d):
    return pl.pallas_call(
        _mm_body,
        grid=(NUM_EMBED // _MM_BLOCK,),
        in_specs=[
            pl.BlockSpec((_MM_BLOCK, EMBED_DIM), lambda i: (i, 0)),
            pl.BlockSpec((EMBED_DIM, HALF), lambda i: (0, 0)),
            pl.BlockSpec((EMBED_DIM, HALF), lambda i: (0, 0)),
            pl.BlockSpec((1, OUT_DIM), lambda i: (0, 0)),
        ],
        out_specs=pl.BlockSpec((_MM_BLOCK, HALF), lambda i: (i, 0)),
        out_shape=jax.ShapeDtypeStruct((NUM_EMBED, HALF), jnp.int32),
    )(emb_table, fc_wt_lo, fc_wt_hi, fc_b2d)


_NW = 32
_ROWS_PER_W = BATCH // _NW
_CHUNK = 100
_CHUNKS_PER_ROW = SEQ // _CHUNK
_CHUNKS_PER_W = _ROWS_PER_W * _CHUNKS_PER_ROW
_NG = HALF // 16
_INV_SEQ = 1.0 / SEQ


def _gather_mean_body(p_hbm, xr_hbm, out_hbm, idx_v, buf_a, buf_b, out_v,
                      sem_a, sem_b):
    wid = lax.axis_index("s") * 2 + lax.axis_index("c")

    pltpu.sync_copy(xr_hbm.at[pl.ds(wid * _CHUNKS_PER_W, _CHUNKS_PER_W)],
                    idx_v)

    def accumulate(buf, acc):
        def body(i, acc):
            acc = list(acc)
            for g in range(_NG):
                packed = buf[i, pl.ds(16 * g, 16)]
                acc[g] = acc[g] + lax.bitcast_convert_type(
                    packed << 16, jnp.float32)
                acc[_NG + g] = acc[_NG + g] + lax.bitcast_convert_type(
                    packed, jnp.float32)
            return tuple(acc)
        return lax.fori_loop(0, _CHUNK, body, acc)

    zeros = tuple(jnp.zeros((16,), jnp.float32) for _ in range(2 * _NG))

    def store_row(row, acc):
        for g in range(_NG):
            out_v[row, pl.ds(16 * g, 16)] = acc[g] * _INV_SEQ
        for g in range(_NG):
            out_v[row, pl.ds(HALF + 16 * g, 16)] = acc[_NG + g] * _INV_SEQ

    pltpu.async_copy(p_hbm.at[idx_v.at[0]], buf_a, sem_a)

    def pair_body(r, _):
        c_a = 2 * r
        c_b = c_a + 1

        pltpu.async_copy(p_hbm.at[idx_v.at[c_b]], buf_b, sem_b)
        pltpu.make_async_copy(p_hbm.at[idx_v.at[c_a]], buf_a, sem_a).wait()
        acc = accumulate(buf_a, zeros)

        @pl.when(r < _ROWS_PER_W - 1)
        def _():
            pltpu.async_copy(p_hbm.at[idx_v.at[c_a + 2]], buf_a, sem_a)

        pltpu.make_async_copy(p_hbm.at[idx_v.at[c_b]], buf_b, sem_b).wait()
        acc = accumulate(buf_b, acc)
        store_row(r, acc)
        return 0

    lax.fori_loop(0, _ROWS_PER_W, pair_body, 0)

    pltpu.sync_copy(out_v, out_hbm.at[pl.ds(wid * _ROWS_PER_W, _ROWS_PER_W)])


def _gather_mean(p, x_chunks):
    mesh = plsc.VectorSubcoreMesh(core_axis_name="c", subcore_axis_name="s")
    run = functools.partial(
        pl.kernel,
        mesh=mesh,
        out_type=jax.ShapeDtypeStruct((BATCH, OUT_DIM), jnp.float32),
        scratch_types=[
            pltpu.VMEM((_CHUNKS_PER_W, _CHUNK), jnp.int32),
            pltpu.VMEM((_CHUNK, HALF), jnp.int32),
            pltpu.VMEM((_CHUNK, HALF), jnp.int32),
            pltpu.VMEM((_ROWS_PER_W, OUT_DIM), jnp.float32),
            pltpu.SemaphoreType.DMA,
            pltpu.SemaphoreType.DMA,
        ],
    )(_gather_mean_body)
    return run(p, x_chunks)


def kernel(x, emb_table, fc_w, fc_b):
    fc_wt = fc_w.T
    p = _project_table(emb_table, fc_wt[:, :HALF], fc_wt[:, HALF:],
                       fc_b.reshape(1, OUT_DIM))
    x_chunks = x.astype(jnp.int32).reshape(BATCH * SEQ // _CHUNK, _CHUNK)
    return _gather_mean(p, x_chunks)


# speedup vs baseline: 15.4113x; 1.2454x over previous
import functools

import jax
import jax.numpy as jnp
from jax import lax
from jax.experimental import pallas as pl
from jax.experimental.pallas import tpu as pltpu
from jax.experimental.pallas import tpu_sc as plsc

NUM_EMBED = 100000
EMBED_DIM = 768
OUT_DIM = 256
HALF = OUT_DIM // 2
BATCH = 4096
SEQ = 200


_MM_BLOCK = 2000


def _pack_bf16_pair(lo, hi):
    lo_bits = lax.bitcast_convert_type(
        lo.astype(jnp.bfloat16).astype(jnp.float32), jnp.int32)
    hi_bits = lax.bitcast_convert_type(
        hi.astype(jnp.bfloat16).astype(jnp.float32), jnp.int32)
    return lax.shift_right_logical(lo_bits, 16) | (
        hi_bits & jnp.int32(-65536))


def _mm_body(e_ref, wlo_ref, whi_ref, b_ref, o_ref):
    e = e_ref[...]
    lo = (jnp.dot(e, wlo_ref[...], preferred_element_type=jnp.float32)
          + b_ref[0:1, :HALF])
    hi = (jnp.dot(e, whi_ref[...], preferred_element_type=jnp.float32)
          + b_ref[0:1, HALF:])
    o_ref[...] = _pack_bf16_pair(lo, hi)


def _project_table(emb_table, fc_wt_lo, fc_wt_hi, fc_b2d):
    return pl.pallas_call(
        _mm_body,
        grid=(NUM_EMBED // _MM_BLOCK,),
        in_specs=[
            pl.BlockSpec((_MM_BLOCK, EMBED_DIM), lambda i: (i, 0)),
            pl.BlockSpec((EMBED_DIM, HALF), lambda i: (0, 0)),
            pl.BlockSpec((EMBED_DIM, HALF), lambda i: (0, 0)),
            pl.BlockSpec((1, OUT_DIM), lambda i: (0, 0)),
        ],
        out_specs=pl.BlockSpec((_MM_BLOCK, HALF), lambda i: (i, 0)),
        out_shape=jax.ShapeDtypeStruct((NUM_EMBED, HALF), jnp.int32),
    )(emb_table, fc_wt_lo, fc_wt_hi, fc_b2d)


_NW = 32
_ROWS_PER_W = BATCH // _NW
_CHUNK = 100
_CHUNKS_PER_ROW = SEQ // _CHUNK
_CHUNKS_PER_W = _ROWS_PER_W * _CHUNKS_PER_ROW
_NG = HALF // 16
_INV_SEQ = 1.0 / SEQ


def _gather_mean_body(p_hbm, xr_hbm, out_hbm, idx_v, buf_a, buf_b, buf_c,
                      buf_d, out_v, sem_a, sem_b, sem_c, sem_d):
    wid = lax.axis_index("s") * 2 + lax.axis_index("c")

    pltpu.sync_copy(xr_hbm.at[pl.ds(wid * _CHUNKS_PER_W, _CHUNKS_PER_W)],
                    idx_v)

    def accumulate(buf, acc):
        def body(i, acc):
            acc = list(acc)
            for g in range(_NG):
                packed = buf[i, pl.ds(16 * g, 16)]
                acc[g] = acc[g] + lax.bitcast_convert_type(
                    packed << 16, jnp.float32)
                acc[_NG + g] = acc[_NG + g] + lax.bitcast_convert_type(
                    packed, jnp.float32)
            return tuple(acc)
        return lax.fori_loop(0, _CHUNK, body, acc)

    zeros = tuple(jnp.zeros((16,), jnp.float32) for _ in range(2 * _NG))

    def store_row(row, acc):
        for g in range(_NG):
            out_v[row, pl.ds(16 * g, 16)] = acc[g] * _INV_SEQ
        for g in range(_NG):
            out_v[row, pl.ds(HALF + 16 * g, 16)] = acc[_NG + g] * _INV_SEQ

    def start(c, buf, sem):
        pltpu.async_copy(p_hbm.at[idx_v.at[c]], buf, sem)

    def wait(c, buf, sem):
        pltpu.make_async_copy(p_hbm.at[idx_v.at[c]], buf, sem).wait()

    start(0, buf_a, sem_a)
    start(1, buf_b, sem_b)
    start(2, buf_c, sem_c)
    start(3, buf_d, sem_d)

    def quad_body(q, _):
        c0 = 4 * q

        wait(c0, buf_a, sem_a)
        acc = accumulate(buf_a, zeros)
        wait(c0 + 1, buf_b, sem_b)
        acc = accumulate(buf_b, acc)
        store_row(2 * q, acc)

        @pl.when(q < _ROWS_PER_W // 2 - 1)
        def _():
            start(c0 + 4, buf_a, sem_a)
            start(c0 + 5, buf_b, sem_b)

        wait(c0 + 2, buf_c, sem_c)
        acc = accumulate(buf_c, zeros)
        wait(c0 + 3, buf_d, sem_d)
        acc = accumulate(buf_d, acc)
        store_row(2 * q + 1, acc)

        @pl.when(q < _ROWS_PER_W // 2 - 1)
        def _():
            start(c0 + 6, buf_c, sem_c)
            start(c0 + 7, buf_d, sem_d)
        return 0

    lax.fori_loop(0, _ROWS_PER_W // 2, quad_body, 0)

    pltpu.sync_copy(out_v, out_hbm.at[pl.ds(wid * _ROWS_PER_W, _ROWS_PER_W)])


def _gather_mean(p, x_chunks):
    mesh = plsc.VectorSubcoreMesh(core_axis_name="c", subcore_axis_name="s")
    run = functools.partial(
        pl.kernel,
        mesh=mesh,
        out_type=jax.ShapeDtypeStruct((BATCH, OUT_DIM), jnp.float32),
        scratch_types=[
            pltpu.VMEM((_CHUNKS_PER_W, _CHUNK), jnp.int32),
            pltpu.VMEM((_CHUNK, HALF), jnp.int32),
            pltpu.VMEM((_CHUNK, HALF), jnp.int32),
            pltpu.VMEM((_CHUNK, HALF), jnp.int32),
            pltpu.VMEM((_CHUNK, HALF), jnp.int32),
            pltpu.VMEM((_ROWS_PER_W, OUT_DIM), jnp.float32),
            pltpu.SemaphoreType.DMA,
            pltpu.SemaphoreType.DMA,
            pltpu.SemaphoreType.DMA,
            pltpu.SemaphoreType.DMA,
        ],
    )(_gather_mean_body)
    return run(p, x_chunks)


def kernel(x, emb_table, fc_w, fc_b):
    fc_wt = fc_w.T
    p = _project_table(emb_table, fc_wt[:, :HALF], fc_wt[:, HALF:],
                       fc_b.reshape(1, OUT_DIM))
    x_chunks = x.astype(jnp.int32).reshape(BATCH * SEQ // _CHUNK, _CHUNK)
    return _gather_mean(p, x_chunks)
